# Initial kernel scaffold; baseline (speedup 1.0000x reference)
import jax
import jax.numpy as jnp
from jax.experimental import pallas as pl


def kernel(h_task, graph_tasks, alpha):
    raise NotImplementedError("write your pallas kernel here")


# speedup vs baseline: 3.4221x; 3.4221x over previous
import jax
import jax.numpy as jnp
from jax.experimental import pallas as pl
from jax.experimental.pallas import tpu as pltpu

_N = 320000
_D = 128
_T = 8
_B = 4000
_NB = _N // _B


def _p1(ids_ref, h_ref, sums_ref, counts_ref):
    step = pl.program_id(0)

    @pl.when(step == 0)
    def _init():
        sums_ref[...] = jnp.zeros_like(sums_ref)
        counts_ref[...] = jnp.zeros_like(counts_ref)

    ids = ids_ref[0]
    tid = jax.lax.broadcasted_iota(jnp.int32, (_T, 1), 0)
    oh = (ids == tid).astype(jnp.float32)
    h = h_ref[...]
    sums_ref[...] += jax.lax.dot_general(
        oh, h, (((1,), (0,)), ((), ())), preferred_element_type=jnp.float32
    )
    counts_ref[...] += jnp.sum(oh, axis=1, keepdims=True)


def _p2(idc_ref, h_ref, sums_ref, counts_ref, acol_ref, out_ref):
    counts = counts_ref[...]
    w = acol_ref[...] * (counts > 0).astype(jnp.float32) / jnp.maximum(counts, 1.0)
    fused = jnp.sum(w * sums_ref[...], axis=0, keepdims=True)
    flag = (idc_ref[...] == 0).astype(jnp.float32)
    out_ref[...] = h_ref[...] + flag * fused


def kernel(h_task, graph_tasks, alpha):
    ids3 = graph_tasks.reshape(_NB, 1, _B)
    idcol = graph_tasks.reshape(_N, 1)
    acol = jnp.transpose(alpha)[:, :1]

    sums, counts = pl.pallas_call(
        _p1,
        grid=(_NB,),
        in_specs=[
            pl.BlockSpec((1, 1, _B), lambda i: (i, 0, 0)),
            pl.BlockSpec((_B, _D), lambda i: (i, 0)),
        ],
        out_specs=[
            pl.BlockSpec((_T, _D), lambda i: (0, 0)),
            pl.BlockSpec((_T, 1), lambda i: (0, 0)),
        ],
        out_shape=[
            jax.ShapeDtypeStruct((_T, _D), jnp.float32),
            jax.ShapeDtypeStruct((_T, 1), jnp.float32),
        ],
    )(ids3, h_task)

    out = pl.pallas_call(
        _p2,
        grid=(_NB,),
        in_specs=[
            pl.BlockSpec((_B, 1), lambda i: (i, 0)),
            pl.BlockSpec((_B, _D), lambda i: (i, 0)),
            pl.BlockSpec((_T, _D), lambda i: (0, 0)),
            pl.BlockSpec((_T, 1), lambda i: (0, 0)),
            pl.BlockSpec((_T, 1), lambda i: (0, 0)),
        ],
        out_specs=pl.BlockSpec((_B, _D), lambda i: (i, 0)),
        out_shape=jax.ShapeDtypeStruct((_N, _D), jnp.float32),
    )(idcol, h_task, sums, counts, acol)
    return out


# speedup vs baseline: 3.7234x; 1.0881x over previous
import jax
import jax.numpy as jnp
from jax.experimental import pallas as pl
from jax.experimental.pallas import tpu as pltpu

_N = 320000
_D = 128
_T = 8
_B = 8000
_NB = _N // _B


def _p1(ids_ref, h_ref, sums_ref, counts_ref):
    step = pl.program_id(0)

    @pl.when(step == 0)
    def _init():
        sums_ref[...] = jnp.zeros_like(sums_ref)
        counts_ref[...] = jnp.zeros_like(counts_ref)

    ids = ids_ref[0]
    tid = jax.lax.broadcasted_iota(jnp.int32, (_T, 1), 0)
    oh = (ids == tid).astype(jnp.float32)
    h = h_ref[...]
    sums_ref[...] += jax.lax.dot_general(
        oh, h, (((1,), (0,)), ((), ())), preferred_element_type=jnp.float32
    )
    counts_ref[...] += jnp.sum(oh, axis=1, keepdims=True)


def _p2(idc_ref, h_ref, sums_ref, counts_ref, acol_ref, out_ref):
    counts = counts_ref[...]
    w = acol_ref[...] * (counts > 0).astype(jnp.float32) / jnp.maximum(counts, 1.0)
    fused = jnp.sum(w * sums_ref[...], axis=0, keepdims=True)
    flag = (idc_ref[...] == 0).astype(jnp.float32)
    out_ref[...] = h_ref[...] + flag * fused


def kernel(h_task, graph_tasks, alpha):
    ids3 = graph_tasks.reshape(_NB, 1, _B)
    idcol = graph_tasks.reshape(_N, 1)
    acol = jnp.transpose(alpha)[:, :1]

    sums, counts = pl.pallas_call(
        _p1,
        grid=(_NB,),
        in_specs=[
            pl.BlockSpec((1, 1, _B), lambda i: (i, 0, 0)),
            pl.BlockSpec((_B, _D), lambda i: (i, 0)),
        ],
        out_specs=[
            pl.BlockSpec((_T, _D), lambda i: (0, 0)),
            pl.BlockSpec((_T, 1), lambda i: (0, 0)),
        ],
        out_shape=[
            jax.ShapeDtypeStruct((_T, _D), jnp.float32),
            jax.ShapeDtypeStruct((_T, 1), jnp.float32),
        ],
    )(ids3, h_task)

    out = pl.pallas_call(
        _p2,
        grid=(_NB,),
        in_specs=[
            pl.BlockSpec((_B, 1), lambda i: (i, 0)),
            pl.BlockSpec((_B, _D), lambda i: (i, 0)),
            pl.BlockSpec((_T, _D), lambda i: (0, 0)),
            pl.BlockSpec((_T, 1), lambda i: (0, 0)),
            pl.BlockSpec((_T, 1), lambda i: (0, 0)),
        ],
        out_specs=pl.BlockSpec((_B, _D), lambda i: (i, 0)),
        out_shape=jax.ShapeDtypeStruct((_N, _D), jnp.float32),
    )(idcol, h_task, sums, counts, acol)
    return out


# speedup vs baseline: 3.7936x; 1.0189x over previous
import jax
import jax.numpy as jnp
from jax.experimental import pallas as pl
from jax.experimental.pallas import tpu as pltpu

_N = 320000
_D = 128
_T = 8
_B = 16000
_NB = _N // _B


def _p1(ids_ref, h_ref, sums_ref, counts_ref):
    step = pl.program_id(0)

    @pl.when(step == 0)
    def _init():
        sums_ref[...] = jnp.zeros_like(sums_ref)
        counts_ref[...] = jnp.zeros_like(counts_ref)

    ids = ids_ref[0]
    tid = jax.lax.broadcasted_iota(jnp.int32, (_T, 1), 0)
    oh = (ids == tid).astype(jnp.float32)
    h = h_ref[...]
    sums_ref[...] += jax.lax.dot_general(
        oh, h, (((1,), (0,)), ((), ())), preferred_element_type=jnp.float32
    )
    counts_ref[...] += jnp.sum(oh, axis=1, keepdims=True)


def _p2(idc_ref, h_ref, sums_ref, counts_ref, acol_ref, out_ref):
    counts = counts_ref[...]
    w = acol_ref[...] * (counts > 0).astype(jnp.float32) / jnp.maximum(counts, 1.0)
    fused = jnp.sum(w * sums_ref[...], axis=0, keepdims=True)
    flag = (idc_ref[...] == 0).astype(jnp.float32)
    out_ref[...] = h_ref[...] + flag * fused


def kernel(h_task, graph_tasks, alpha):
    ids3 = graph_tasks.reshape(_NB, 1, _B)
    idcol = graph_tasks.reshape(_N, 1)
    acol = jnp.transpose(alpha)[:, :1]

    sums, counts = pl.pallas_call(
        _p1,
        grid=(_NB,),
        in_specs=[
            pl.BlockSpec((1, 1, _B), lambda i: (i, 0, 0)),
            pl.BlockSpec((_B, _D), lambda i: (i, 0)),
        ],
        out_specs=[
            pl.BlockSpec((_T, _D), lambda i: (0, 0)),
            pl.BlockSpec((_T, 1), lambda i: (0, 0)),
        ],
        out_shape=[
            jax.ShapeDtypeStruct((_T, _D), jnp.float32),
            jax.ShapeDtypeStruct((_T, 1), jnp.float32),
        ],
    )(ids3, h_task)

    out = pl.pallas_call(
        _p2,
        grid=(_NB,),
        in_specs=[
            pl.BlockSpec((_B, 1), lambda i: (i, 0)),
            pl.BlockSpec((_B, _D), lambda i: (i, 0)),
            pl.BlockSpec((_T, _D), lambda i: (0, 0)),
            pl.BlockSpec((_T, 1), lambda i: (0, 0)),
            pl.BlockSpec((_T, 1), lambda i: (0, 0)),
        ],
        out_specs=pl.BlockSpec((_B, _D), lambda i: (i, 0)),
        out_shape=jax.ShapeDtypeStruct((_N, _D), jnp.float32),
    )(idcol, h_task, sums, counts, acol)
    return out


# speedup vs baseline: 4.7571x; 1.2540x over previous
import functools

import jax
import jax.numpy as jnp
from jax import lax
from jax.experimental import pallas as pl
from jax.experimental.pallas import tpu as pltpu
from jax.experimental.pallas import tpu_sc as plsc

_N = 320000
_D = 128
_T = 8
_B = 16000
_NB = _N // _B

_NC = 2
_NS = 16
_NW = _NC * _NS
_RPW = _N // _NW
_G16 = _RPW // 16
_C = 128
_IDXW = _RPW + _C + 16


def _pA(ids_ref, h_ref, acol_ref, out_ref, fused_ref, sums, counts):
    step = pl.program_id(0)

    @pl.when(step == 0)
    def _init():
        sums[...] = jnp.zeros_like(sums)
        counts[...] = jnp.zeros_like(counts)

    ids = ids_ref[0]
    tid = lax.broadcasted_iota(jnp.int32, (_T, 1), 0)
    oh = (ids == tid).astype(jnp.float32)
    h = h_ref[...]
    out_ref[...] = h
    sums[...] += lax.dot_general(
        oh, h, (((1,), (0,)), ((), ())), preferred_element_type=jnp.float32
    )
    counts[...] += jnp.sum(oh, axis=1, keepdims=True)

    @pl.when(step == _NB - 1)
    def _fold():
        c = counts[...]
        w = acol_ref[...] * (c > 0).astype(jnp.float32) / jnp.maximum(c, 1.0)
        fused_ref[...] = jnp.sum(w * sums[...], axis=0, keepdims=True)


def _pB(out_ref, ids_hbm, fused_hbm, idsbuf, shiftb, rowsb, idxc, fbuf, rowbuf, shidx, sem):
    cid = lax.axis_index("c")
    sid = lax.axis_index("s")
    wid = sid * _NC + cid
    base = wid * _RPW
    sbase = sid * _IDXW
    dump = _RPW + _C
    pltpu.sync_copy(ids_hbm.at[pl.ds(base, _RPW)], idsbuf)
    pltpu.sync_copy(fused_hbm, fbuf)

    shiftb[pl.ds(0, 16)] = jnp.zeros((16,), jnp.int32)

    def g_body(g, cur):
        v = idsbuf[pl.ds(g * 16, 16)]
        m = v == 0
        mi = jnp.where(m, 1, 0)
        lane = lax.iota(jnp.int32, 16)
        rows = lane + (base + g * 16)
        shiftb[pl.ds(16, 16)] = mi
        ps = mi + shiftb[pl.ds(15, 16)]
        shiftb[pl.ds(16, 16)] = ps
        ps = ps + shiftb[pl.ds(14, 16)]
        shiftb[pl.ds(16, 16)] = ps
        ps = ps + shiftb[pl.ds(12, 16)]
        shiftb[pl.ds(16, 16)] = ps
        ps = ps + shiftb[pl.ds(8, 16)]
        pos = sbase + jnp.where(m, cur + ps - 1, dump + lane)
        rowsb[pl.ds(0, 16)] = rows
        pltpu.async_copy(rowsb, shidx.at[pos], sem).wait()
        return cur + ps[15]

    cur = lax.fori_loop(0, _G16, g_body, 0)
    nchunks = (cur + _C - 1) // _C

    def c_body(k, _):
        pltpu.sync_copy(shidx.at[pl.ds(sbase + k * _C, _C)], idxc)
        rem = jnp.minimum(cur - k * _C, _C)
        firstv = idxc[pl.ds(0, 16)][0]
        for j in range(_C // 16):
            w = idxc[pl.ds(j * 16, 16)]
            lanepos = lax.iota(jnp.int32, 16) + j * 16
            idxc[pl.ds(j * 16, 16)] = jnp.where(
                lanepos < rem, w, jnp.full((16,), 0, jnp.int32) + firstv
            )
        pltpu.async_copy(out_ref.at[idxc], rowbuf, sem).wait()

        def r_body(i, _):
            for d in range(8):
                rowbuf[i, pl.ds(d * 16, 16)] += fbuf[pl.ds(d * 16, 16)]
            return 0

        lax.fori_loop(0, _C, r_body, 0)
        pltpu.async_copy(rowbuf, out_ref.at[idxc], sem).wait()
        return 0

    lax.fori_loop(0, nchunks, c_body, 0)


_fixup = functools.partial(
    pl.kernel,
    mesh=plsc.VectorSubcoreMesh(core_axis_name="c", subcore_axis_name="s"),
    scratch_types=[
        pltpu.VMEM((_RPW,), jnp.int32),
        pltpu.VMEM((32,), jnp.int32),
        pltpu.VMEM((16,), jnp.int32),
        pltpu.VMEM((_C,), jnp.int32),
        pltpu.VMEM((_D,), jnp.float32),
        pltpu.VMEM((_C, _D), jnp.float32),
        pltpu.VMEM_SHARED((_NS * _IDXW,), jnp.int32),
        pltpu.SemaphoreType.DMA,
    ],
)(_pB, out_type=())


def kernel(h_task, graph_tasks, alpha):
    ids3 = graph_tasks.reshape(_NB, 1, _B)
    acol = jnp.transpose(alpha)[:, :1]

    out_a, fused = pl.pallas_call(
        _pA,
        grid=(_NB,),
        in_specs=[
            pl.BlockSpec((1, 1, _B), lambda i: (i, 0, 0)),
            pl.BlockSpec((_B, _D), lambda i: (i, 0)),
            pl.BlockSpec((_T, 1), lambda i: (0, 0)),
        ],
        out_specs=[
            pl.BlockSpec((_B, _D), lambda i: (i, 0)),
            pl.BlockSpec((1, _D), lambda i: (0, 0)),
        ],
        out_shape=[
            jax.ShapeDtypeStruct((_N, _D), jnp.float32),
            jax.ShapeDtypeStruct((1, _D), jnp.float32),
        ],
        scratch_shapes=[
            pltpu.VMEM((_T, _D), jnp.float32),
            pltpu.VMEM((_T, 1), jnp.float32),
        ],
    )(ids3, h_task, acol)

    acc = jax.new_ref(out_a)
    _fixup(acc, graph_tasks, fused.reshape(_D))
    return jax.freeze(acc)


# speedup vs baseline: 5.3466x; 1.1239x over previous
import functools

import jax
import jax.numpy as jnp
from jax import lax
from jax.experimental import pallas as pl
from jax.experimental.pallas import tpu as pltpu
from jax.experimental.pallas import tpu_sc as plsc

_N = 320000
_D = 128
_T = 8
_B = 16000
_NB = _N // _B

_NC = 2
_NS = 16
_NW = _NC * _NS
_RPW = _N // _NW
_G16 = _RPW // 16
_C = 128
_IDXW = _RPW + _C + 128


def _pA(ids_ref, h_ref, acol_ref, out_ref, fused_ref, sums, counts):
    step = pl.program_id(0)

    @pl.when(step == 0)
    def _init():
        sums[...] = jnp.zeros_like(sums)
        counts[...] = jnp.zeros_like(counts)

    ids = ids_ref[0]
    tid = lax.broadcasted_iota(jnp.int32, (_T, 1), 0)
    oh = (ids == tid).astype(jnp.float32)
    h = h_ref[...]
    out_ref[...] = h
    sums[...] += lax.dot_general(
        oh, h, (((1,), (0,)), ((), ())), preferred_element_type=jnp.float32
    )
    counts[...] += jnp.sum(oh, axis=1, keepdims=True)

    @pl.when(step == _NB - 1)
    def _fold():
        c = counts[...]
        w = acol_ref[...] * (c > 0).astype(jnp.float32) / jnp.maximum(c, 1.0)
        fused_ref[...] = jnp.sum(w * sums[...], axis=0, keepdims=True)


def _pB(out_ref, ids_hbm, fused_hbm, idsbuf, shiftb, rowsb, posb, idxc, fbuf, rowbuf, shidx, sem):
    cid = lax.axis_index("c")
    sid = lax.axis_index("s")
    wid = sid * _NC + cid
    base = wid * _RPW
    sbase = sid * _IDXW
    dump = _RPW + _C
    pltpu.sync_copy(ids_hbm.at[pl.ds(base, _RPW)], idsbuf)
    pltpu.sync_copy(fused_hbm, fbuf)

    shiftb[pl.ds(0, 16)] = jnp.zeros((16,), jnp.int32)

    def _group(g, cur, t):
        v = idsbuf[pl.ds(g * 16, 16)]
        m = v == 0
        mi = jnp.where(m, 1, 0)
        lane = lax.iota(jnp.int32, 16)
        rows = lane + (base + g * 16)
        shiftb[pl.ds(16, 16)] = mi
        ps = mi + shiftb[pl.ds(15, 16)]
        shiftb[pl.ds(16, 16)] = ps
        ps = ps + shiftb[pl.ds(14, 16)]
        shiftb[pl.ds(16, 16)] = ps
        ps = ps + shiftb[pl.ds(12, 16)]
        shiftb[pl.ds(16, 16)] = ps
        ps = ps + shiftb[pl.ds(8, 16)]
        pos = sbase + jnp.where(m, cur + ps - 1, dump + t * 16 + lane)
        rowsb[pl.ds(t * 16, 16)] = rows
        posb[pl.ds(t * 16, 16)] = pos
        return cur + ps[15]

    def b_body(b, cur):
        for t in range(8):
            cur = _group(b * 8 + t, cur, t)
        pltpu.async_copy(rowsb, shidx.at[posb], sem).wait()
        return cur

    cur = lax.fori_loop(0, _G16 // 8, b_body, 0)
    cur = _group(_G16 - 1, cur, 0)
    pltpu.async_copy(rowsb, shidx.at[posb], sem).wait()
    nchunks = (cur + _C - 1) // _C

    def c_body(k, _):
        pltpu.sync_copy(shidx.at[pl.ds(sbase + k * _C, _C)], idxc)
        rem = jnp.minimum(cur - k * _C, _C)
        firstv = idxc[pl.ds(0, 16)][0]
        for j in range(_C // 16):
            w = idxc[pl.ds(j * 16, 16)]
            lanepos = lax.iota(jnp.int32, 16) + j * 16
            idxc[pl.ds(j * 16, 16)] = jnp.where(
                lanepos < rem, w, jnp.full((16,), 0, jnp.int32) + firstv
            )
        pltpu.async_copy(out_ref.at[idxc], rowbuf, sem).wait()

        def r_body(i, _):
            for d in range(8):
                rowbuf[i, pl.ds(d * 16, 16)] += fbuf[pl.ds(d * 16, 16)]
            return 0

        lax.fori_loop(0, _C, r_body, 0)
        pltpu.async_copy(rowbuf, out_ref.at[idxc], sem).wait()
        return 0

    lax.fori_loop(0, nchunks, c_body, 0)


_fixup = functools.partial(
    pl.kernel,
    mesh=plsc.VectorSubcoreMesh(core_axis_name="c", subcore_axis_name="s"),
    scratch_types=[
        pltpu.VMEM((_RPW,), jnp.int32),
        pltpu.VMEM((32,), jnp.int32),
        pltpu.VMEM((_C,), jnp.int32),
        pltpu.VMEM((_C,), jnp.int32),
        pltpu.VMEM((_C,), jnp.int32),
        pltpu.VMEM((_D,), jnp.float32),
        pltpu.VMEM((_C, _D), jnp.float32),
        pltpu.VMEM_SHARED((_NS * _IDXW,), jnp.int32),
        pltpu.SemaphoreType.DMA,
    ],
)(_pB, out_type=())


def kernel(h_task, graph_tasks, alpha):
    ids3 = graph_tasks.reshape(_NB, 1, _B)
    acol = jnp.transpose(alpha)[:, :1]

    out_a, fused = pl.pallas_call(
        _pA,
        grid=(_NB,),
        in_specs=[
            pl.BlockSpec((1, 1, _B), lambda i: (i, 0, 0)),
            pl.BlockSpec((_B, _D), lambda i: (i, 0)),
            pl.BlockSpec((_T, 1), lambda i: (0, 0)),
        ],
        out_specs=[
            pl.BlockSpec((_B, _D), lambda i: (i, 0)),
            pl.BlockSpec((1, _D), lambda i: (0, 0)),
        ],
        out_shape=[
            jax.ShapeDtypeStruct((_N, _D), jnp.float32),
            jax.ShapeDtypeStruct((1, _D), jnp.float32),
        ],
        scratch_shapes=[
            pltpu.VMEM((_T, _D), jnp.float32),
            pltpu.VMEM((_T, 1), jnp.float32),
        ],
    )(ids3, h_task, acol)

    acc = jax.new_ref(out_a)
    _fixup(acc, graph_tasks, fused.reshape(_D))
    return jax.freeze(acc)


# speedup vs baseline: 5.7244x; 1.0707x over previous
import functools

import jax
import jax.numpy as jnp
from jax import lax
from jax.experimental import pallas as pl
from jax.experimental.pallas import tpu as pltpu
from jax.experimental.pallas import tpu_sc as plsc

_N = 320000
_D = 128
_T = 8
_B = 16000
_NB = _N // _B

_NC = 2
_NS = 16
_NW = _NC * _NS
_RPW = _N // _NW
_G16 = _RPW // 16
_C = 128
_IDXW = _RPW + _C + 128


def _pA(ids_ref, h_ref, acol_ref, out_ref, fused_ref, sums, counts):
    step = pl.program_id(0)

    @pl.when(step == 0)
    def _init():
        sums[...] = jnp.zeros_like(sums)
        counts[...] = jnp.zeros_like(counts)

    ids = ids_ref[0]
    tid = lax.broadcasted_iota(jnp.int32, (_T, 1), 0)
    oh = (ids == tid).astype(jnp.float32)
    h = h_ref[...]
    out_ref[...] = h
    sums[...] += lax.dot_general(
        oh, h, (((1,), (0,)), ((), ())), preferred_element_type=jnp.float32
    )
    counts[...] += jnp.sum(oh, axis=1, keepdims=True)

    @pl.when(step == _NB - 1)
    def _fold():
        c = counts[...]
        w = acol_ref[...] * (c > 0).astype(jnp.float32) / jnp.maximum(c, 1.0)
        fused_ref[...] = jnp.sum(w * sums[...], axis=0, keepdims=True)


def _pB1(ids_hbm, idx_hbm, cnt_hbm, idsbuf, shiftb, rowsb, posb, wbuf, shidx, sem):
    cid = lax.axis_index("c")
    sid = lax.axis_index("s")
    wid = sid * _NC + cid
    base = wid * _RPW
    sbase = sid * _IDXW
    dump = _RPW + _C
    pltpu.sync_copy(ids_hbm.at[pl.ds(base, _RPW)], idsbuf)

    shiftb[pl.ds(0, 16)] = jnp.zeros((16,), jnp.int32)

    def _group(g, cur, t):
        v = idsbuf[pl.ds(g * 16, 16)]
        m = v == 0
        mi = jnp.where(m, 1, 0)
        lane = lax.iota(jnp.int32, 16)
        rows = lane + (base + g * 16)
        shiftb[pl.ds(16, 16)] = mi
        ps = mi + shiftb[pl.ds(15, 16)]
        shiftb[pl.ds(16, 16)] = ps
        ps = ps + shiftb[pl.ds(14, 16)]
        shiftb[pl.ds(16, 16)] = ps
        ps = ps + shiftb[pl.ds(12, 16)]
        shiftb[pl.ds(16, 16)] = ps
        ps = ps + shiftb[pl.ds(8, 16)]
        pos = sbase + jnp.where(m, cur + ps - 1, dump + t * 16 + lane)
        rowsb[pl.ds(t * 16, 16)] = rows
        posb[pl.ds(t * 16, 16)] = pos
        return cur + ps[15]

    def b_body(b, cur):
        for t in range(8):
            cur = _group(b * 8 + t, cur, t)
        pltpu.async_copy(rowsb, shidx.at[posb], sem).wait()
        return cur

    cur = lax.fori_loop(0, _G16 // 8, b_body, 0)
    cur = _group(_G16 - 1, cur, 0)
    pltpu.async_copy(rowsb, shidx.at[posb], sem).wait()

    pltpu.sync_copy(shidx.at[pl.ds(sbase, _IDXW)], wbuf)
    pltpu.sync_copy(wbuf, idx_hbm.at[pl.ds(wid * _IDXW, _IDXW)])
    rowsb[pl.ds(0, 16)] = jnp.full((16,), 0, jnp.int32) + cur
    pltpu.sync_copy(rowsb.at[pl.ds(0, 16)], cnt_hbm.at[pl.ds(wid * 16, 16)])


def _pB2(out_ref, idx_hbm, cnt_hbm, fused_hbm, cntbuf, idxc, fbuf, rowbuf, sem):
    cid = lax.axis_index("c")
    sid = lax.axis_index("s")
    wid = sid * _NC + cid
    pltpu.sync_copy(cnt_hbm.at[pl.ds(wid * 16, 16)], cntbuf)
    pltpu.sync_copy(fused_hbm, fbuf)
    cur = cntbuf[pl.ds(0, 16)][0]
    nchunks = (cur + _C - 1) // _C

    def c_body(k, _):
        pltpu.sync_copy(idx_hbm.at[pl.ds(wid * _IDXW + k * _C, _C)], idxc)
        rem = jnp.minimum(cur - k * _C, _C)
        firstv = idxc[pl.ds(0, 16)][0]
        for j in range(_C // 16):
            w = idxc[pl.ds(j * 16, 16)]
            lanepos = lax.iota(jnp.int32, 16) + j * 16
            idxc[pl.ds(j * 16, 16)] = jnp.where(
                lanepos < rem, w, jnp.full((16,), 0, jnp.int32) + firstv
            )
        pltpu.async_copy(out_ref.at[idxc], rowbuf, sem).wait()

        def r_body(i, _):
            for d in range(8):
                rowbuf[i, pl.ds(d * 16, 16)] += fbuf[pl.ds(d * 16, 16)]
            return 0

        lax.fori_loop(0, _C, r_body, 0)
        pltpu.async_copy(rowbuf, out_ref.at[idxc], sem).wait()
        return 0

    lax.fori_loop(0, nchunks, c_body, 0)


_scan_compact = pl.kernel(
    _pB1,
    out_type=[
        jax.ShapeDtypeStruct((_NW * _IDXW,), jnp.int32),
        jax.ShapeDtypeStruct((_NW * 16,), jnp.int32),
    ],
    mesh=plsc.VectorSubcoreMesh(core_axis_name="c", subcore_axis_name="s"),
    scratch_types=[
        pltpu.VMEM((_RPW,), jnp.int32),
        pltpu.VMEM((32,), jnp.int32),
        pltpu.VMEM((_C,), jnp.int32),
        pltpu.VMEM((_C,), jnp.int32),
        pltpu.VMEM((_IDXW,), jnp.int32),
        pltpu.VMEM_SHARED((_NS * _IDXW,), jnp.int32),
        pltpu.SemaphoreType.DMA,
    ],
)

_apply_fused = pl.kernel(
    _pB2,
    out_type=(),
    mesh=plsc.VectorSubcoreMesh(core_axis_name="c", subcore_axis_name="s"),
    scratch_types=[
        pltpu.VMEM((16,), jnp.int32),
        pltpu.VMEM((_C,), jnp.int32),
        pltpu.VMEM((_D,), jnp.float32),
        pltpu.VMEM((_C, _D), jnp.float32),
        pltpu.SemaphoreType.DMA,
    ],
)


def kernel(h_task, graph_tasks, alpha):
    ids3 = graph_tasks.reshape(_NB, 1, _B)
    acol = jnp.transpose(alpha)[:, :1]

    idxs, cnts = _scan_compact(graph_tasks)

    out_a, fused = pl.pallas_call(
        _pA,
        grid=(_NB,),
        in_specs=[
            pl.BlockSpec((1, 1, _B), lambda i: (i, 0, 0)),
            pl.BlockSpec((_B, _D), lambda i: (i, 0)),
            pl.BlockSpec((_T, 1), lambda i: (0, 0)),
        ],
        out_specs=[
            pl.BlockSpec((_B, _D), lambda i: (i, 0)),
            pl.BlockSpec((1, _D), lambda i: (0, 0)),
        ],
        out_shape=[
            jax.ShapeDtypeStruct((_N, _D), jnp.float32),
            jax.ShapeDtypeStruct((1, _D), jnp.float32),
        ],
        scratch_shapes=[
            pltpu.VMEM((_T, _D), jnp.float32),
            pltpu.VMEM((_T, 1), jnp.float32),
        ],
    )(ids3, h_task, acol)

    acc = jax.new_ref(out_a)
    _apply_fused(acc, idxs, cnts, fused.reshape(_D))
    return jax.freeze(acc)


# speedup vs baseline: 5.7253x; 1.0002x over previous
import functools

import jax
import jax.numpy as jnp
from jax import lax
from jax.experimental import pallas as pl
from jax.experimental.pallas import tpu as pltpu
from jax.experimental.pallas import tpu_sc as plsc

_N = 320000
_D = 128
_T = 8
_B = 16000
_NB = _N // _B

_NC = 2
_NS = 16
_NW = _NC * _NS
_RPW = _N // _NW
_G16 = _RPW // 16
_C = 128
_IDXW = _RPW + _C + 128
_NBAT = _G16 // 8 + 1


def _pA(ids_ref, h_ref, acol_ref, out_ref, fused_ref, sums, counts):
    step = pl.program_id(0)

    @pl.when(step == 0)
    def _init():
        sums[...] = jnp.zeros_like(sums)
        counts[...] = jnp.zeros_like(counts)

    ids = ids_ref[0]
    tid = lax.broadcasted_iota(jnp.int32, (_T, 1), 0)
    oh = (ids == tid).astype(jnp.float32)
    h = h_ref[...]
    out_ref[...] = h
    sums[...] += lax.dot_general(
        oh, h, (((1,), (0,)), ((), ())), preferred_element_type=jnp.float32
    )
    counts[...] += jnp.sum(oh, axis=1, keepdims=True)

    @pl.when(step == _NB - 1)
    def _fold():
        c = counts[...]
        w = acol_ref[...] * (c > 0).astype(jnp.float32) / jnp.maximum(c, 1.0)
        fused_ref[...] = jnp.sum(w * sums[...], axis=0, keepdims=True)


def _pB1(ids_hbm, idx_hbm, cnt_hbm, idsbuf, shiftb, rowsb, posb, wbuf, shidx, sem):
    cid = lax.axis_index("c")
    sid = lax.axis_index("s")
    wid = sid * _NC + cid
    base = wid * _RPW
    sbase = sid * _IDXW
    dump = _RPW + _C
    pltpu.sync_copy(ids_hbm.at[pl.ds(base, _RPW)], idsbuf)

    shiftb[pl.ds(0, 16)] = jnp.zeros((16,), jnp.int32)

    def _group(b, g, cur, t):
        v = idsbuf[pl.ds(g * 16, 16)]
        m = v == 0
        mi = jnp.where(m, 1, 0)
        lane = lax.iota(jnp.int32, 16)
        rows = lane + (base + g * 16)
        shiftb[pl.ds(16, 16)] = mi
        ps = mi + shiftb[pl.ds(15, 16)]
        shiftb[pl.ds(16, 16)] = ps
        ps = ps + shiftb[pl.ds(14, 16)]
        shiftb[pl.ds(16, 16)] = ps
        ps = ps + shiftb[pl.ds(12, 16)]
        shiftb[pl.ds(16, 16)] = ps
        ps = ps + shiftb[pl.ds(8, 16)]
        pos = sbase + jnp.where(m, cur + ps - 1, dump + t * 16 + lane)
        rowsb[b, pl.ds(t * 16, 16)] = rows
        posb[b, pl.ds(t * 16, 16)] = pos
        return cur + ps[15]

    def b_body(b, cur):
        for t in range(8):
            cur = _group(b, b * 8 + t, cur, t)
        return cur

    cur = lax.fori_loop(0, _NBAT - 1, b_body, 0)
    cur = _group(_NBAT - 1, _G16 - 1, cur, 0)
    for t in range(1, 8):
        lane = lax.iota(jnp.int32, 16)
        posb[_NBAT - 1, pl.ds(t * 16, 16)] = sbase + dump + t * 16 + lane
        rowsb[_NBAT - 1, pl.ds(t * 16, 16)] = lane

    def fire(b, _):
        pltpu.async_copy(rowsb.at[b], shidx.at[posb.at[b]], sem)

        @pl.when(b >= 8)
        def _():
            pltpu.make_async_copy(rowsb.at[0], shidx.at[posb.at[0]], sem).wait()

        return 0

    lax.fori_loop(0, _NBAT, fire, 0)

    def drain(b, _):
        pltpu.make_async_copy(rowsb.at[0], shidx.at[posb.at[0]], sem).wait()
        return 0

    lax.fori_loop(0, 8, drain, 0)

    pltpu.sync_copy(shidx.at[pl.ds(sbase, _IDXW)], wbuf)
    pltpu.sync_copy(wbuf, idx_hbm.at[pl.ds(wid * _IDXW, _IDXW)])
    shiftb[pl.ds(16, 16)] = jnp.full((16,), 0, jnp.int32) + cur
    pltpu.sync_copy(shiftb.at[pl.ds(16, 16)], cnt_hbm.at[pl.ds(wid * 16, 16)])


def _pB2(out_ref, idx_hbm, cnt_hbm, fused_hbm, cntbuf, idxc, fbuf, rowbuf, sem):
    cid = lax.axis_index("c")
    sid = lax.axis_index("s")
    wid = sid * _NC + cid
    pltpu.sync_copy(cnt_hbm.at[pl.ds(wid * 16, 16)], cntbuf)
    pltpu.sync_copy(fused_hbm, fbuf)
    cur = cntbuf[pl.ds(0, 16)][0]
    nchunks = (cur + _C - 1) // _C

    def c_body(k, _):
        pltpu.sync_copy(idx_hbm.at[pl.ds(wid * _IDXW + k * _C, _C)], idxc)
        rem = jnp.minimum(cur - k * _C, _C)
        firstv = idxc[pl.ds(0, 16)][0]
        for j in range(_C // 16):
            w = idxc[pl.ds(j * 16, 16)]
            lanepos = lax.iota(jnp.int32, 16) + j * 16
            idxc[pl.ds(j * 16, 16)] = jnp.where(
                lanepos < rem, w, jnp.full((16,), 0, jnp.int32) + firstv
            )
        pltpu.async_copy(out_ref.at[idxc], rowbuf, sem).wait()

        def r_body(i, _):
            for d in range(8):
                rowbuf[i, pl.ds(d * 16, 16)] += fbuf[pl.ds(d * 16, 16)]
            return 0

        lax.fori_loop(0, _C, r_body, 0)
        pltpu.async_copy(rowbuf, out_ref.at[idxc], sem).wait()
        return 0

    lax.fori_loop(0, nchunks, c_body, 0)


_scan_compact = pl.kernel(
    _pB1,
    out_type=[
        jax.ShapeDtypeStruct((_NW * _IDXW,), jnp.int32),
        jax.ShapeDtypeStruct((_NW * 16,), jnp.int32),
    ],
    mesh=plsc.VectorSubcoreMesh(core_axis_name="c", subcore_axis_name="s"),
    scratch_types=[
        pltpu.VMEM((_RPW,), jnp.int32),
        pltpu.VMEM((32,), jnp.int32),
        pltpu.VMEM((_NBAT, _C), jnp.int32),
        pltpu.VMEM((_NBAT, _C), jnp.int32),
        pltpu.VMEM((_IDXW,), jnp.int32),
        pltpu.VMEM_SHARED((_NS * _IDXW,), jnp.int32),
        pltpu.SemaphoreType.DMA,
    ],
)

_apply_fused = pl.kernel(
    _pB2,
    out_type=(),
    mesh=plsc.VectorSubcoreMesh(core_axis_name="c", subcore_axis_name="s"),
    scratch_types=[
        pltpu.VMEM((16,), jnp.int32),
        pltpu.VMEM((_C,), jnp.int32),
        pltpu.VMEM((_D,), jnp.float32),
        pltpu.VMEM((_C, _D), jnp.float32),
        pltpu.SemaphoreType.DMA,
    ],
)


def kernel(h_task, graph_tasks, alpha):
    ids3 = graph_tasks.reshape(_NB, 1, _B)
    acol = jnp.transpose(alpha)[:, :1]

    idxs, cnts = _scan_compact(graph_tasks)

    out_a, fused = pl.pallas_call(
        _pA,
        grid=(_NB,),
        in_specs=[
            pl.BlockSpec((1, 1, _B), lambda i: (i, 0, 0)),
            pl.BlockSpec((_B, _D), lambda i: (i, 0)),
            pl.BlockSpec((_T, 1), lambda i: (0, 0)),
        ],
        out_specs=[
            pl.BlockSpec((_B, _D), lambda i: (i, 0)),
            pl.BlockSpec((1, _D), lambda i: (0, 0)),
        ],
        out_shape=[
            jax.ShapeDtypeStruct((_N, _D), jnp.float32),
            jax.ShapeDtypeStruct((1, _D), jnp.float32),
        ],
        scratch_shapes=[
            pltpu.VMEM((_T, _D), jnp.float32),
            pltpu.VMEM((_T, 1), jnp.float32),
        ],
    )(ids3, h_task, acol)

    acc = jax.new_ref(out_a)
    _apply_fused(acc, idxs, cnts, fused.reshape(_D))
    return jax.freeze(acc)


# speedup vs baseline: 5.7407x; 1.0027x over previous
import functools

import jax
import jax.numpy as jnp
from jax import lax
from jax.experimental import pallas as pl
from jax.experimental.pallas import tpu as pltpu
from jax.experimental.pallas import tpu_sc as plsc

_N = 320000
_D = 128
_T = 8
_B = 16000
_NB = _N // _B

_NC = 2
_NS = 16
_NW = _NC * _NS
_RPW = _N // _NW
_G16 = _RPW // 16
_C = 128
_IDXW = _RPW + _C + 128
_NBAT = _G16 // 8 + 1


def _pA(ids_ref, h_ref, acol_ref, out_ref, fused_ref, sums, counts):
    step = pl.program_id(0)

    @pl.when(step == 0)
    def _init():
        sums[...] = jnp.zeros_like(sums)
        counts[...] = jnp.zeros_like(counts)

    ids = ids_ref[0]
    tid = lax.broadcasted_iota(jnp.int32, (_T, 1), 0)
    oh = (ids == tid).astype(jnp.float32)
    h = h_ref[...]
    out_ref[...] = h
    sums[...] += lax.dot_general(
        oh, h, (((1,), (0,)), ((), ())), preferred_element_type=jnp.float32
    )
    counts[...] += jnp.sum(oh, axis=1, keepdims=True)

    @pl.when(step == _NB - 1)
    def _fold():
        c = counts[...]
        w = acol_ref[...] * (c > 0).astype(jnp.float32) / jnp.maximum(c, 1.0)
        fused_ref[...] = jnp.sum(w * sums[...], axis=0, keepdims=True)


def _pB1(ids_hbm, idx_hbm, cnt_hbm, idsbuf, shiftb, rowsb, posb, wbuf, shidx, sem):
    cid = lax.axis_index("c")
    sid = lax.axis_index("s")
    wid = sid * _NC + cid
    base = wid * _RPW
    sbase = sid * _IDXW
    dump = _RPW + _C
    pltpu.sync_copy(ids_hbm.at[pl.ds(base, _RPW)], idsbuf)

    for z in range(16):
        shiftb[pl.ds(z * 16, 16)] = jnp.zeros((16,), jnp.int32)

    def _batch(b, cur, nt):
        lane = lax.iota(jnp.int32, 16)
        mis = []
        for t in range(nt):
            g = b * 8 + t
            v = idsbuf[pl.ds(g * 16, 16)]
            mi = jnp.where(v == 0, 1, 0)
            mis.append(mi)
            rowsb[b, pl.ds(t * 16, 16)] = lane + (base + g * 16)
        pss = list(mis)
        for k in (1, 2, 4, 8):
            for t in range(nt):
                shiftb[pl.ds(t * 32 + 16, 16)] = pss[t]
            for t in range(nt):
                pss[t] = pss[t] + shiftb[pl.ds(t * 32 + 16 - k, 16)]
        for t in range(nt):
            pos = sbase + jnp.where(
                mis[t] == 1, cur + pss[t] - 1, dump + t * 16 + lane
            )
            posb[b, pl.ds(t * 16, 16)] = pos
            cur = cur + pss[t][15]
        return cur

    def b_body(b, cur):
        return _batch(b, cur, 8)

    cur = lax.fori_loop(0, _NBAT - 1, b_body, 0)
    cur = _batch(_NBAT - 1, cur, 1)
    for t in range(1, 8):
        lane = lax.iota(jnp.int32, 16)
        posb[_NBAT - 1, pl.ds(t * 16, 16)] = sbase + dump + t * 16 + lane
        rowsb[_NBAT - 1, pl.ds(t * 16, 16)] = lane

    def fire(b, _):
        pltpu.async_copy(rowsb.at[b], shidx.at[posb.at[b]], sem)

        @pl.when(b >= 8)
        def _():
            pltpu.make_async_copy(rowsb.at[0], shidx.at[posb.at[0]], sem).wait()

        return 0

    lax.fori_loop(0, _NBAT, fire, 0)

    def drain(b, _):
        pltpu.make_async_copy(rowsb.at[0], shidx.at[posb.at[0]], sem).wait()
        return 0

    lax.fori_loop(0, 8, drain, 0)

    pltpu.sync_copy(shidx.at[pl.ds(sbase, _IDXW)], wbuf)
    pltpu.sync_copy(wbuf, idx_hbm.at[pl.ds(wid * _IDXW, _IDXW)])
    shiftb[pl.ds(16, 16)] = jnp.full((16,), 0, jnp.int32) + cur
    pltpu.sync_copy(shiftb.at[pl.ds(16, 16)], cnt_hbm.at[pl.ds(wid * 16, 16)])


def _pB2(out_ref, idx_hbm, cnt_hbm, fused_hbm, cntbuf, idxc, fbuf, rowbuf, sem):
    cid = lax.axis_index("c")
    sid = lax.axis_index("s")
    wid = sid * _NC + cid
    pltpu.sync_copy(cnt_hbm.at[pl.ds(wid * 16, 16)], cntbuf)
    pltpu.sync_copy(fused_hbm, fbuf)
    cur = cntbuf[pl.ds(0, 16)][0]
    nchunks = (cur + _C - 1) // _C

    def c_body(k, _):
        pltpu.sync_copy(idx_hbm.at[pl.ds(wid * _IDXW + k * _C, _C)], idxc)
        rem = jnp.minimum(cur - k * _C, _C)
        firstv = idxc[pl.ds(0, 16)][0]
        for j in range(_C // 16):
            w = idxc[pl.ds(j * 16, 16)]
            lanepos = lax.iota(jnp.int32, 16) + j * 16
            idxc[pl.ds(j * 16, 16)] = jnp.where(
                lanepos < rem, w, jnp.full((16,), 0, jnp.int32) + firstv
            )
        pltpu.async_copy(out_ref.at[idxc], rowbuf, sem).wait()

        def r_body(i, _):
            for d in range(8):
                rowbuf[i, pl.ds(d * 16, 16)] += fbuf[pl.ds(d * 16, 16)]
            return 0

        lax.fori_loop(0, _C, r_body, 0)
        pltpu.async_copy(rowbuf, out_ref.at[idxc], sem).wait()
        return 0

    lax.fori_loop(0, nchunks, c_body, 0)


_scan_compact = pl.kernel(
    _pB1,
    out_type=[
        jax.ShapeDtypeStruct((_NW * _IDXW,), jnp.int32),
        jax.ShapeDtypeStruct((_NW * 16,), jnp.int32),
    ],
    mesh=plsc.VectorSubcoreMesh(core_axis_name="c", subcore_axis_name="s"),
    scratch_types=[
        pltpu.VMEM((_RPW,), jnp.int32),
        pltpu.VMEM((256,), jnp.int32),
        pltpu.VMEM((_NBAT, _C), jnp.int32),
        pltpu.VMEM((_NBAT, _C), jnp.int32),
        pltpu.VMEM((_IDXW,), jnp.int32),
        pltpu.VMEM_SHARED((_NS * _IDXW,), jnp.int32),
        pltpu.SemaphoreType.DMA,
    ],
)

_apply_fused = pl.kernel(
    _pB2,
    out_type=(),
    mesh=plsc.VectorSubcoreMesh(core_axis_name="c", subcore_axis_name="s"),
    scratch_types=[
        pltpu.VMEM((16,), jnp.int32),
        pltpu.VMEM((_C,), jnp.int32),
        pltpu.VMEM((_D,), jnp.float32),
        pltpu.VMEM((_C, _D), jnp.float32),
        pltpu.SemaphoreType.DMA,
    ],
)


def kernel(h_task, graph_tasks, alpha):
    ids3 = graph_tasks.reshape(_NB, 1, _B)
    acol = jnp.transpose(alpha)[:, :1]

    idxs, cnts = _scan_compact(graph_tasks)

    out_a, fused = pl.pallas_call(
        _pA,
        grid=(_NB,),
        in_specs=[
            pl.BlockSpec((1, 1, _B), lambda i: (i, 0, 0)),
            pl.BlockSpec((_B, _D), lambda i: (i, 0)),
            pl.BlockSpec((_T, 1), lambda i: (0, 0)),
        ],
        out_specs=[
            pl.BlockSpec((_B, _D), lambda i: (i, 0)),
            pl.BlockSpec((1, _D), lambda i: (0, 0)),
        ],
        out_shape=[
            jax.ShapeDtypeStruct((_N, _D), jnp.float32),
            jax.ShapeDtypeStruct((1, _D), jnp.float32),
        ],
        scratch_shapes=[
            pltpu.VMEM((_T, _D), jnp.float32),
            pltpu.VMEM((_T, 1), jnp.float32),
        ],
    )(ids3, h_task, acol)

    acc = jax.new_ref(out_a)
    _apply_fused(acc, idxs, cnts, fused.reshape(_D))
    return jax.freeze(acc)
